# Initial kernel scaffold; baseline (speedup 1.0000x reference)
import functools

import jax
import jax.numpy as jnp
from jax import lax
from jax.experimental import pallas as pl
from jax.experimental.pallas import tpu as pltpu
from jax.experimental.pallas import tpu_sc as plsc

DIM = 32
B = 4096
LC = 200
LQ = 20

NC = 2
NS = 16
NW = NC * NS

SUB = 128
K = 10
CHUNK = SUB * K

N_CTX = B * LC
N_Q = B * LQ


def _body(word_c, pos_c, kg_c, word_q, pos_q, kg_q,
          W_word, W_pos, W_kg,
          ctx_out, q_out,
          idx_v, rows_v, gsem):
  wid = lax.axis_index("s") * NC + lax.axis_index("c")

  jobs = (
      (W_word, word_c, ctx_out, 0 * N_CTX, N_CTX),
      (W_pos, pos_c, ctx_out, 1 * N_CTX, N_CTX),
      (W_kg, kg_c, ctx_out, 2 * N_CTX, N_CTX),
      (W_word, word_q, q_out, 0 * N_Q, N_Q),
      (W_pos, pos_q, q_out, 1 * N_Q, N_Q),
      (W_kg, kg_q, q_out, 2 * N_Q, N_Q),
  )

  for table, idx2d, out, out_base, n_rows in jobs:
    per_w = n_rows // NW
    n_chunks = per_w // CHUNK
    assert per_w % CHUNK == 0
    idx_row0 = wid * (per_w // SUB)
    out0 = out_base + wid * per_w

    @pl.loop(0, n_chunks)
    def _(i, table=table, idx2d=idx2d, out=out, idx_row0=idx_row0, out0=out0):
      pltpu.sync_copy(idx2d.at[pl.ds(idx_row0 + i * K, K)], idx_v)
      copies = []
      for j in range(K):
        copies.append(
            pltpu.async_copy(table.at[idx_v.at[j]],
                             rows_v.at[pl.ds(j * SUB, SUB)], gsem))
      for c in copies:
        c.wait()
      pltpu.sync_copy(rows_v, out.at[pl.ds(out0 + i * CHUNK, CHUNK)])


@jax.jit
def _run(word_c, pos_c, kg_c, word_q, pos_q, kg_q, W_word, W_pos, W_kg):
  mesh = plsc.VectorSubcoreMesh(core_axis_name="c", subcore_axis_name="s")
  ctx_flat, q_flat = pl.kernel(
      _body,
      out_type=(
          jax.ShapeDtypeStruct((3 * N_CTX, DIM), jnp.float32),
          jax.ShapeDtypeStruct((3 * N_Q, DIM), jnp.float32),
      ),
      mesh=mesh,
      scratch_types=[
          pltpu.VMEM((K, SUB), jnp.int32),
          pltpu.VMEM((CHUNK, DIM), jnp.float32),
          pltpu.SemaphoreType.DMA,
      ],
  )(word_c, pos_c, kg_c, word_q, pos_q, kg_q, W_word, W_pos, W_kg)
  return ctx_flat, q_flat


def kernel(word_context, word_question, kg_context, kg_question,
           pos_context, pos_question, W_word, W_pos, W_kg):
  ctx_flat, q_flat = _run(
      word_context.reshape(N_CTX // SUB, SUB),
      pos_context.reshape(N_CTX // SUB, SUB),
      kg_context.reshape(N_CTX // SUB, SUB),
      word_question.reshape(N_Q // SUB, SUB),
      pos_question.reshape(N_Q // SUB, SUB),
      kg_question.reshape(N_Q // SUB, SUB),
      W_word, W_pos, W_kg)
  return (ctx_flat.reshape(3 * B, LC, DIM), q_flat.reshape(3 * B, LQ, DIM))


# speedup vs baseline: 2.6961x; 2.6961x over previous
import functools

import jax
import jax.numpy as jnp
from jax import lax
from jax.experimental import pallas as pl
from jax.experimental.pallas import tpu as pltpu
from jax.experimental.pallas import tpu_sc as plsc

DIM = 32
B = 4096
LC = 200
LQ = 20

NC = 2
NS = 16
NW = NC * NS

SUB = 128
K = 8
CHUNK = SUB * K

N_CTX = B * LC
N_Q = B * LQ


def _body(word_c, pos_c, kg_c, word_q, pos_q, kg_q,
          W_word, W_pos, W_kg,
          ctx_out, q_out,
          idx_v, rows_v, gsem):
  wid = lax.axis_index("s") * NC + lax.axis_index("c")

  jobs = (
      (W_word, word_c, ctx_out, 0 * N_CTX, N_CTX),
      (W_pos, pos_c, ctx_out, 1 * N_CTX, N_CTX),
      (W_kg, kg_c, ctx_out, 2 * N_CTX, N_CTX),
      (W_word, word_q, q_out, 0 * N_Q, N_Q),
      (W_pos, pos_q, q_out, 1 * N_Q, N_Q),
      (W_kg, kg_q, q_out, 2 * N_Q, N_Q),
  )

  for table, idx3d, out, out_base, n_rows in jobs:
    nb = n_rows // CHUNK
    assert n_rows % CHUNK == 0

    @pl.loop(wid, nb, step=NW)
    def _(g, table=table, idx3d=idx3d, out=out, out_base=out_base):
      pltpu.sync_copy(idx3d.at[g], idx_v)
      copies = []
      for j in range(K):
        copies.append(
            pltpu.async_copy(table.at[idx_v.at[j]],
                             rows_v.at[pl.ds(j * SUB, SUB)], gsem))
      for c in copies:
        c.wait()
      pltpu.sync_copy(rows_v, out.at[pl.ds(out_base + g * CHUNK, CHUNK)])


@jax.jit
def _run(word_c, pos_c, kg_c, word_q, pos_q, kg_q, W_word, W_pos, W_kg):
  mesh = plsc.VectorSubcoreMesh(core_axis_name="c", subcore_axis_name="s")
  ctx_flat, q_flat = pl.kernel(
      _body,
      out_type=(
          jax.ShapeDtypeStruct((3 * N_CTX, DIM), jnp.float32),
          jax.ShapeDtypeStruct((3 * N_Q, DIM), jnp.float32),
      ),
      mesh=mesh,
      compiler_params=pltpu.CompilerParams(use_tc_tiling_on_sc=False),
      scratch_types=[
          pltpu.VMEM((K, SUB), jnp.int32),
          pltpu.VMEM((CHUNK, DIM), jnp.float32),
          pltpu.SemaphoreType.DMA,
      ],
  )(word_c, pos_c, kg_c, word_q, pos_q, kg_q, W_word, W_pos, W_kg)
  return ctx_flat, q_flat


def kernel(word_context, word_question, kg_context, kg_question,
           pos_context, pos_question, W_word, W_pos, W_kg):
  ctx_flat, q_flat = _run(
      word_context.reshape(N_CTX // CHUNK, K, SUB),
      pos_context.reshape(N_CTX // CHUNK, K, SUB),
      kg_context.reshape(N_CTX // CHUNK, K, SUB),
      word_question.reshape(N_Q // CHUNK, K, SUB),
      pos_question.reshape(N_Q // CHUNK, K, SUB),
      kg_question.reshape(N_Q // CHUNK, K, SUB),
      W_word, W_pos, W_kg)
  return (ctx_flat.reshape(3 * B, LC, DIM), q_flat.reshape(3 * B, LQ, DIM))


# speedup vs baseline: 2.7428x; 1.0173x over previous
import functools

import jax
import jax.numpy as jnp
from jax import lax
from jax.experimental import pallas as pl
from jax.experimental.pallas import tpu as pltpu
from jax.experimental.pallas import tpu_sc as plsc

DIM = 32
B = 4096
LC = 200
LQ = 20

NC = 2
NS = 16
NW = NC * NS

SUB = 128
KC = 8
KQ = 4

N_CTX = B * LC
N_Q = B * LQ
NI_CTX = N_CTX // (KC * SUB) // NW
NI_Q = N_Q // (KQ * SUB) // NW


def _body(word_c, pos_c, kg_c, word_q, pos_q, kg_q,
          W_word, W_pos, W_kg,
          ctx_out, q_out,
          idx0, idx1, rows0, rows1, sem0, sem1):
  wid = lax.axis_index("s") * NC + lax.axis_index("c")
  idx = (idx0, idx1)
  rows = (rows0, rows1)
  sem = (sem0, sem1)

  jobs = (
      (W_word, word_c, ctx_out, 0 * N_CTX, KC, NI_CTX),
      (W_pos, pos_c, ctx_out, 1 * N_CTX, KC, NI_CTX),
      (W_kg, kg_c, ctx_out, 2 * N_CTX, KC, NI_CTX),
      (W_word, word_q, q_out, 0 * N_Q, KQ, NI_Q),
      (W_pos, pos_q, q_out, 1 * N_Q, KQ, NI_Q),
      (W_kg, kg_q, q_out, 2 * N_Q, KQ, NI_Q),
  )

  for table, idx3d, out, out_base, K, n_i in jobs:
    ch = K * SUB
    first = wid * n_i

    def fire(i, p, table=table, idx3d=idx3d, K=K, first=first):
      pltpu.sync_copy(idx3d.at[first + i], idx[p].at[pl.ds(0, K)])
      for j in range(K):
        pltpu.async_copy(table.at[idx[p].at[j]],
                         rows[p].at[pl.ds(j * SUB, SUB)], sem[p])

    def drain(i, p, table=table, out=out, out_base=out_base, K=K, ch=ch,
              first=first):
      for j in range(K):
        pltpu.make_async_copy(table.at[idx[p].at[j]],
                              rows[p].at[pl.ds(j * SUB, SUB)], sem[p]).wait()
      pltpu.sync_copy(rows[p].at[pl.ds(0, ch)],
                      out.at[pl.ds(out_base + (first + i) * ch, ch)])

    fire(0, 0)

    @pl.loop(0, (n_i - 1) // 2)
    def _(t, fire=fire, drain=drain):
      fire(2 * t + 1, 1)
      drain(2 * t, 0)
      fire(2 * t + 2, 0)
      drain(2 * t + 1, 1)

    drain(n_i - 1, 0)


@jax.jit
def _run(word_c, pos_c, kg_c, word_q, pos_q, kg_q, W_word, W_pos, W_kg):
  mesh = plsc.VectorSubcoreMesh(core_axis_name="c", subcore_axis_name="s")
  ctx_flat, q_flat = pl.kernel(
      _body,
      out_type=(
          jax.ShapeDtypeStruct((3 * N_CTX, DIM), jnp.float32),
          jax.ShapeDtypeStruct((3 * N_Q, DIM), jnp.float32),
      ),
      mesh=mesh,
      compiler_params=pltpu.CompilerParams(use_tc_tiling_on_sc=False),
      scratch_types=[
          pltpu.VMEM((KC, SUB), jnp.int32),
          pltpu.VMEM((KC, SUB), jnp.int32),
          pltpu.VMEM((KC * SUB, DIM), jnp.float32),
          pltpu.VMEM((KC * SUB, DIM), jnp.float32),
          pltpu.SemaphoreType.DMA,
          pltpu.SemaphoreType.DMA,
      ],
  )(word_c, pos_c, kg_c, word_q, pos_q, kg_q, W_word, W_pos, W_kg)
  return ctx_flat, q_flat


def kernel(word_context, word_question, kg_context, kg_question,
           pos_context, pos_question, W_word, W_pos, W_kg):
  ctx_flat, q_flat = _run(
      word_context.reshape(-1, KC, SUB),
      pos_context.reshape(-1, KC, SUB),
      kg_context.reshape(-1, KC, SUB),
      word_question.reshape(-1, KQ, SUB),
      pos_question.reshape(-1, KQ, SUB),
      kg_question.reshape(-1, KQ, SUB),
      W_word, W_pos, W_kg)
  return (ctx_flat.reshape(3 * B, LC, DIM), q_flat.reshape(3 * B, LQ, DIM))
